# baseline (speedup 1.0000x reference)
import functools

import jax
import jax.numpy as jnp
from jax import lax
from jax.experimental import pallas as pl
from jax.experimental.pallas import tpu as pltpu, tpu_sc as plsc

N_NODES = 10000
N_EDGES = 160000
DIM = 384
NUM_HEADS = 8
NEG_SLOPE = 0.2

NLANE = 16
NSUB = 16
NCORE = 2
NW = NSUB * NCORE


def _mm_heads_body(x_ref, w_ref, o_ref):
    o_ref[0] = jnp.dot(x_ref[...], w_ref[0],
                       preferred_element_type=jnp.float32)


def _mm_heads(x, w_hmajor, bm=1000):
    n, k = x.shape
    h, _, d = w_hmajor.shape
    return pl.pallas_call(
        _mm_heads_body,
        grid=(h, n // bm),
        in_specs=[
            pl.BlockSpec((bm, k), lambda hh, i: (i, 0)),
            pl.BlockSpec((1, k, d), lambda hh, i: (hh, 0, 0)),
        ],
        out_specs=pl.BlockSpec((1, bm, d), lambda hh, i: (hh, i, 0)),
        out_shape=jax.ShapeDtypeStruct((h, n, d), jnp.float32),
    )(x, w_hmajor)


def _max_body(x_ref, o_ref):
    i = pl.program_id(0)

    @pl.when(i == 0)
    def _():
        o_ref[...] = jnp.full_like(o_ref, -jnp.inf)

    m = jnp.max(x_ref[...], axis=1, keepdims=True)
    o_ref[...] = jnp.maximum(o_ref[...], jnp.broadcast_to(m, o_ref.shape))


def _exp_body(x_ref, m_ref, o_ref):
    m = m_ref[...][:, 0:1]
    o_ref[...] = jnp.exp(x_ref[...] - m)


def _tc_softmax_prep(logits, be=6400):
    h, e = logits.shape
    m = pl.pallas_call(
        _max_body,
        grid=(e // be,),
        in_specs=[pl.BlockSpec((h, be), lambda i: (0, i))],
        out_specs=pl.BlockSpec((h, 128), lambda i: (0, 0)),
        out_shape=jax.ShapeDtypeStruct((h, 128), jnp.float32),
    )(logits)
    return pl.pallas_call(
        _exp_body,
        grid=(e // be,),
        in_specs=[
            pl.BlockSpec((h, be), lambda i: (0, i)),
            pl.BlockSpec((h, 128), lambda i: (0, 0)),
        ],
        out_specs=pl.BlockSpec((h, be), lambda i: (0, i)),
        out_shape=jax.ShapeDtypeStruct((h, e), jnp.float32),
    )(logits, m)


def _elu_mm_body(s_ref, wl_ref, wr_ref, ol_ref, or_ref):
    h = s_ref.shape[0]
    accl = jnp.zeros(ol_ref.shape, jnp.float32)
    accr = jnp.zeros(or_ref.shape, jnp.float32)
    for hh in range(h):
        v = s_ref[hh]
        g = jnp.where(v > 0.0, v, jnp.exp(v) - 1.0)
        accl = accl + jnp.dot(g, wl_ref[hh], preferred_element_type=jnp.float32)
        accr = accr + jnp.dot(g, wr_ref[hh], preferred_element_type=jnp.float32)
    ol_ref[...] = accl
    or_ref[...] = accr


def _tc_elu_mm(s, wl, wr, bm=1000):
    h, n, d = s.shape
    return pl.pallas_call(
        _elu_mm_body,
        grid=(n // bm,),
        in_specs=[
            pl.BlockSpec((h, bm, d), lambda i: (0, i, 0)),
            pl.BlockSpec((h, d, d), lambda i: (0, 0, 0)),
            pl.BlockSpec((h, d, d), lambda i: (0, 0, 0)),
        ],
        out_specs=[
            pl.BlockSpec((bm, d), lambda i: (i, 0)),
            pl.BlockSpec((bm, d), lambda i: (i, 0)),
        ],
        out_shape=[
            jax.ShapeDtypeStruct((n, d), jnp.float32),
            jax.ShapeDtypeStruct((n, d), jnp.float32),
        ],
    )(s, wl, wr)


E_PAD = 160768
_E_PER_W = E_PAD // NW
_GB = 32


def _sc_logits_build(num_heads):
    nb = _E_PER_W // _GB

    mesh = plsc.VectorSubcoreMesh(core_axis_name="c", subcore_axis_name="s")

    @functools.partial(
        pl.kernel,
        mesh=mesh,
        compiler_params=pltpu.CompilerParams(needs_layout_passes=False),
        out_type=jax.ShapeDtypeStruct((num_heads * E_PAD,), jnp.float32),
        scratch_types=[
            pltpu.VMEM((_E_PER_W,), jnp.int32),
            pltpu.VMEM((_E_PER_W,), jnp.int32),
            pltpu.VMEM((_E_PER_W,), jnp.int32),
            pltpu.VMEM((_E_PER_W,), jnp.int32),
            pltpu.VMEM((_GB, DIM), jnp.float32),
            pltpu.VMEM((_GB, DIM), jnp.float32),
            pltpu.VMEM((256,), jnp.float32),
            pltpu.VMEM((_E_PER_W,), jnp.float32),
            pltpu.VMEM((num_heads, DIM), jnp.float32),
            pltpu.SemaphoreType.DMA,
            pltpu.SemaphoreType.DMA,
        ],
    )
    def k(hs_hbm, hd_hbm, src_hbm, dst_hbm, a_hbm, out_hbm,
          src_b, dst_b, gs_b, gd_b, hs_r, hd_r, stash, lg_b, a_b,
          sem1, sem2):
        wid = lax.axis_index("c") * NSUB + lax.axis_index("s")
        pltpu.sync_copy(a_hbm, a_b)
        iota = lax.iota(jnp.int32, NLANE)
        base = wid * _E_PER_W
        pltpu.sync_copy(src_hbm.at[pl.ds(base, _E_PER_W)], src_b)
        pltpu.sync_copy(dst_hbm.at[pl.ds(base, _E_PER_W)], dst_b)

        def head_loop(h, _):
            a_vregs = [a_b[h, pl.ds(j * NLANE, NLANE)]
                       for j in range(DIM // NLANE)]
            hoff = h * N_NODES

            def idx_loop(i, _):
                sl = pl.ds(i * NLANE, NLANE)
                gs_b[sl] = src_b[sl] + hoff
                gd_b[sl] = dst_b[sl] + hoff
                return 0

            lax.fori_loop(0, _E_PER_W // NLANE, idx_loop, 0)

            def batch_loop(b, _):
                start = b * _GB
                cp1 = pltpu.async_copy(
                    hs_hbm.at[gs_b.at[pl.ds(start, _GB)]], hs_r, sem1)
                cp2 = pltpu.async_copy(
                    hd_hbm.at[gd_b.at[pl.ds(start, _GB)]], hd_r, sem2)
                cp1.wait()
                cp2.wait()
                for g in range(_GB // NLANE):
                    for i in range(NLANE):
                        r = g * NLANE + i
                        acc = jnp.zeros((NLANE,), jnp.float32)
                        for j in range(DIM // NLANE):
                            sl = pl.ds(j * NLANE, NLANE)
                            z = hs_r[r, sl] + hd_r[r, sl]
                            l = jnp.maximum(z, NEG_SLOPE * z)
                            acc = acc + l * a_vregs[j]
                        stash[pl.ds(i * NLANE, NLANE)] = acc
                    colsum = jnp.zeros((NLANE,), jnp.float32)
                    for j in range(NLANE):
                        colsum = colsum + plsc.load_gather(
                            stash, [iota * NLANE + j])
                    lg_b[pl.ds(start + g * NLANE, NLANE)] = colsum
                return 0

            lax.fori_loop(0, nb, batch_loop, 0)
            pltpu.sync_copy(lg_b,
                            out_hbm.at[pl.ds(h * E_PAD + base, _E_PER_W)])
            return 0

        lax.fori_loop(0, num_heads, head_loop, 0)

    return k


_sc_logits_h8 = _sc_logits_build(NUM_HEADS)
_sc_logits_h1 = _sc_logits_build(1)


_QS = 2512
_QLAST = N_NODES - 3 * _QS
_NQ_PAD = 2560
_TR = _NQ_PAD // NSUB
_EPT = E_PAD // NSUB
_SUP_C = 2512
_NSUP_C = _EPT // _SUP_C
_OCAP = 2560
_RB = 64


def _sc_aggregate_build(num_heads):
    mesh = plsc.VectorSubcoreMesh(core_axis_name="c", subcore_axis_name="s")

    @functools.partial(
        pl.kernel,
        mesh=mesh,
        compiler_params=pltpu.CompilerParams(needs_layout_passes=False),
        out_type=jax.ShapeDtypeStruct((num_heads * N_NODES, DIM), jnp.float32),
        scratch_types=[
            pltpu.VMEM((_SUP_C,), jnp.int32),
            pltpu.VMEM((_SUP_C,), jnp.int32),
            pltpu.VMEM((_SUP_C,), jnp.float32),
            pltpu.VMEM((_OCAP,), jnp.int32),
            pltpu.VMEM((_OCAP,), jnp.int32),
            pltpu.VMEM((_OCAP,), jnp.float32),
            pltpu.VMEM((_RB,), jnp.int32),
            pltpu.VMEM((_RB,), jnp.float32),
            pltpu.VMEM((_RB, DIM), jnp.float32),
            pltpu.VMEM((NLANE, DIM), jnp.float32),
            pltpu.VMEM((32, DIM), jnp.float32),
            pltpu.VMEM((_TR,), jnp.float32),
            pltpu.VMEM((_TR,), jnp.float32),
            pltpu.VMEM_SHARED((_NQ_PAD, DIM), jnp.float32),
            pltpu.VMEM_SHARED((_NQ_PAD,), jnp.float32),
            pltpu.SemaphoreType.DMA,
        ],
    )
    def k(hs_hbm, ex_hbm, src_hbm, dst_hbm, out_hbm,
          src_b, dst_b, exe_b, osrc, odst, oex, odst_b, oex_b, rows,
          rows_f, zrows, zden, invd, acc, denm, sem):
        cid = lax.axis_index("c")
        sid = lax.axis_index("s")
        tbase = sid * _EPT

        zv = jnp.zeros((NLANE,), jnp.float32)

        def zrow_loop(i, _):
            for j in range(DIM // NLANE):
                zrows[i, pl.ds(j * NLANE, NLANE)] = zv
            return 0

        lax.fori_loop(0, 32, zrow_loop, 0)
        for q in range(_TR // NLANE):
            zden[pl.ds(q * NLANE, NLANE)] = zv

        def head_loop(h, _):
            hoff = h * N_NODES

            for q4 in range(2):
                qid = cid * 2 + q4
                lo = qid * _QS
                qsize = jnp.where(qid == 3, _QLAST, _QS)
                hi = lo + qsize

                for q in range(_TR // 32):
                    pltpu.sync_copy(
                        zrows, acc.at[pl.ds(sid * _TR + q * 32, 32)])
                pltpu.sync_copy(zden, denm.at[pl.ds(sid * _TR, _TR)])
                plsc.subcore_barrier()

                def super_loop(sc2, _):
                    ebase = tbase + sc2 * _SUP_C
                    pltpu.sync_copy(src_hbm.at[pl.ds(ebase, _SUP_C)], src_b)
                    pltpu.sync_copy(dst_hbm.at[pl.ds(ebase, _SUP_C)], dst_b)
                    pltpu.sync_copy(
                        ex_hbm.at[pl.ds(h * E_PAD + ebase, _SUP_C)], exe_b)

                    def zown_loop(i, _):
                        sl = pl.ds(i * NLANE, NLANE)
                        osrc[sl] = jnp.zeros((NLANE,), jnp.int32)
                        odst[sl] = jnp.zeros((NLANE,), jnp.int32)
                        oex[sl] = zv
                        return 0

                    lax.fori_loop(0, _OCAP // NLANE, zown_loop, 0)

                    def group_loop(g, off):
                        sl = pl.ds(g * NLANE, NLANE)
                        dstv = dst_b[sl]
                        srcv = src_b[sl]
                        exv = exe_b[sl]
                        own = (dstv >= lo) & (dstv < hi)
                        cum = plsc.cumsum(jnp.where(own, 1, 0))
                        pos = off + cum - 1
                        plsc.store_scatter(osrc, [pos], srcv + hoff, mask=own)
                        plsc.store_scatter(odst, [pos], dstv - lo, mask=own)
                        plsc.store_scatter(oex, [pos], exv, mask=own)
                        return off + jnp.max(cum)

                    off = lax.fori_loop(0, _SUP_C // NLANE, group_loop, 0)
                    nb2 = (off + _RB - 1) // _RB

                    def batch_loop(b, _):
                        bs = b * _RB
                        pltpu.async_copy(
                            hs_hbm.at[osrc.at[pl.ds(bs, _RB)]], rows,
                            sem).wait()
                        for i in range(_RB // NLANE):
                            odst_b[pl.ds(i * NLANE, NLANE)] = (
                                odst[pl.ds(bs + i * NLANE, NLANE)])
                            oex_b[pl.ds(i * NLANE, NLANE)] = (
                                oex[pl.ds(bs + i * NLANE, NLANE)])
                        for r in range(_RB):
                            spl = plsc.load_gather(
                                oex_b, [jnp.full((NLANE,), r, jnp.int32)])
                            for j in range(DIM // NLANE):
                                rsl = pl.ds(j * NLANE, NLANE)
                                rows[r, rsl] = rows[r, rsl] * spl
                        pltpu.sync_copy(rows, acc.at[odst_b.at[pl.ds(0, _RB)]], add=True)
                        pltpu.sync_copy(oex_b, denm.at[odst_b.at[pl.ds(0, _RB)]], add=True)
                        return 0

                    lax.fori_loop(0, nb2, batch_loop, 0)
                    return 0

                lax.fori_loop(0, _NSUP_C, super_loop, 0)
                plsc.subcore_barrier()

                pltpu.sync_copy(denm.at[pl.ds(sid * _TR, _TR)], invd)
                for q in range(_TR // NLANE):
                    sl = pl.ds(q * NLANE, NLANE)
                    invd[sl] = 1.0 / (invd[sl] + 1e-9)

                for q in range(_TR // NLANE):
                    r0 = sid * _TR + q * NLANE

                    @pl.when(r0 + NLANE <= qsize)
                    def _():
                        pltpu.sync_copy(acc.at[pl.ds(r0, NLANE)], rows_f)
                        for r in range(NLANE):
                            spl = plsc.load_gather(
                                invd,
                                [jnp.full((NLANE,), q * NLANE + r, jnp.int32)])
                            for j in range(DIM // NLANE):
                                rsl = pl.ds(j * NLANE, NLANE)
                                rows_f[r, rsl] = rows_f[r, rsl] * spl
                        pltpu.sync_copy(
                            rows_f, out_hbm.at[pl.ds(hoff + lo + r0, NLANE)])

                plsc.subcore_barrier()
            return 0

        lax.fori_loop(0, num_heads, head_loop, 0)

    return k


_sc_aggregate_h8 = _sc_aggregate_build(NUM_HEADS)
_sc_aggregate_h1 = _sc_aggregate_build(1)


def _gat_layer(x_or_hs, src, dst, a, num_heads, hs=None, hd=None):
    sc_logits = _sc_logits_h8 if num_heads == NUM_HEADS else _sc_logits_h1
    hs_flat = hs.reshape(num_heads * N_NODES, DIM)
    hd_flat = hd.reshape(num_heads * N_NODES, DIM)
    logits = sc_logits(hs_flat, hd_flat, src, dst, a)
    logits = logits.reshape(num_heads, E_PAD)[:, :N_EDGES]
    ex = _tc_softmax_prep(logits)
    srcu = src[:N_EDGES]
    dstu = dst[:N_EDGES]
    denom = jax.ops.segment_sum(ex.T, dstu, num_segments=N_NODES)
    msg = ex.T[:, :, None] * hs.transpose(1, 0, 2)[srcu]
    out = jax.ops.segment_sum(msg, dstu, num_segments=N_NODES)
    out = out / (denom[:, :, None] + 1e-9)
    return out.transpose(1, 0, 2)


def kernel(x, edge_index, Wl1, Wr1, a1, Wl2, Wr2, a2):
    pad = E_PAD - N_EDGES
    src = jnp.concatenate(
        [edge_index[0].astype(jnp.int32), jnp.zeros((pad,), jnp.int32)])
    dst = jnp.concatenate(
        [edge_index[1].astype(jnp.int32), jnp.zeros((pad,), jnp.int32)])

    wl1 = Wl1.reshape(DIM, NUM_HEADS, DIM).transpose(1, 0, 2)
    wr1 = Wr1.reshape(DIM, NUM_HEADS, DIM).transpose(1, 0, 2)
    hs1 = _mm_heads(x, wl1)
    hd1 = _mm_heads(x, wr1)
    s1 = _gat_layer(x, src, dst, a1, NUM_HEADS, hs=hs1, hd=hd1)

    wl2 = Wl2.reshape(NUM_HEADS, DIM, DIM)
    wr2 = Wr2.reshape(NUM_HEADS, DIM, DIM)
    hs2, hd2 = _tc_elu_mm(s1, wl2, wr2)
    s2 = _gat_layer(None, src, dst, a2, 1,
                    hs=hs2[None], hd=hd2[None])
    return s2[0]
